# baseline (speedup 1.0000x reference)
import functools

import numpy as np
import jax
import jax.numpy as jnp
from jax import lax
from jax.experimental import pallas as pl
from jax.experimental.pallas import tpu as pltpu
from jax.experimental.pallas import tpu_sc as plsc

_DIAGONAL_OFFSET = 2
_L = 16


def _rc_order_np(ut_len: int, diagonal_offset: int) -> np.ndarray:
    seq_len = int(np.sqrt(2 * ut_len + 0.25) - 0.5) + diagonal_offset
    ut_indexes = np.triu_indices(seq_len, diagonal_offset)
    mat_ut_indexes = np.zeros((seq_len, seq_len), dtype=np.int64)
    mat_ut_indexes[ut_indexes] = np.arange(ut_len)
    mask_ut = np.zeros((seq_len, seq_len), dtype=bool)
    mask_ut[ut_indexes] = True
    mat_indexes = mat_ut_indexes + np.multiply(~mask_ut, mat_ut_indexes.T)
    return mat_indexes[::-1, ::-1][ut_indexes].astype(np.int32)


@functools.lru_cache(maxsize=4)
def _build(B: int, U: int, D: int):
    NC, NS = 2, 16
    BPC = B // NC
    U8 = -(-U // 8) * 8
    CH = (U // NS) // 8 * 8
    REM = U - NS * CH
    P = 128
    sizes = [P] * (CH // P) + ([CH % P] if CH % P else [])
    np_ = len(sizes)
    REM_BASE = NS * CH
    U_PAD = U if REM == 0 else REM_BASE + _L
    G = D // _L

    mesh = plsc.VectorSubcoreMesh(core_axis_name="c", subcore_axis_name="s")

    @functools.partial(
        pl.kernel,
        mesh=mesh,
        out_type=jax.ShapeDtypeStruct((B, U, D), jnp.float32),
        scratch_types=[
            pltpu.HBM((B * U8, 2 * D), jnp.float32),
            pltpu.VMEM((CH,), jnp.int32),
            pltpu.VMEM((P, D), jnp.float32),
            pltpu.VMEM((P, D), jnp.float32),
            pltpu.VMEM((P, 2 * D), jnp.float32),
            pltpu.VMEM((P, 2 * D), jnp.float32),
            pltpu.VMEM((P, 2 * D), jnp.float32),
            pltpu.VMEM((P, D), jnp.float32),
            pltpu.VMEM((_L,), jnp.int32),
            pltpu.SemaphoreType.DMA,
            pltpu.SemaphoreType.DMA,
            pltpu.SemaphoreType.DMA,
            pltpu.SemaphoreType.DMA,
        ],
    )
    def sc_gather(x_hbm, rc_hbm, rev_hbm, out_hbm,
                  lin_hbm, idx_v, la0, la1, lb, gb0, gb1, ob, rev_v,
                  seml0, seml1, semg0, semg1):
        cid = lax.axis_index("c")
        sid = lax.axis_index("s")
        base = sid * CH
        las, gbs = (la0, la1), (gb0, gb1)
        semls, semgs = (seml0, seml1), (semg0, semg1)

        pltpu.sync_copy(rev_hbm, rev_v)
        m = rev_v[...] != 0
        lane = lax.iota(jnp.int32, _L)

        def startl(b, p):
            size = sizes[p]
            return pltpu.async_copy(
                x_hbm.at[b, pl.ds(base + p * P, size), :],
                las[p % 2].at[pl.ds(0, size)], semls[p % 2])

        def repack(src, dst, size):
            def body(t, c):
                for g in range(G):
                    dst[t, pl.ds(g * _L, _L)] = src[t, pl.ds(g * _L, _L)]
                return c
            lax.fori_loop(0, size, body, 0)

        def l_piece(b, p, copies):
            copies[("l", p)].wait()
            size = sizes[p]
            if p + 2 < np_:
                copies[("l", p + 2)] = startl(b, p + 2)
            repack(las[p % 2], lb, size)
            pltpu.sync_copy(lb.at[pl.ds(0, size)],
                            lin_hbm.at[pl.ds(b * U8 + base + p * P, size)])

        def l_rem(b):
            @pl.when(sid == NS - 1)
            def _():
                cp = pltpu.async_copy(
                    x_hbm.at[b, pl.ds(REM_BASE, REM), :],
                    las[0].at[pl.ds(0, REM)], semls[0])
                cp.wait()
                repack(las[0], lb, REM)
                pltpu.sync_copy(lb.at[pl.ds(0, REM)],
                                lin_hbm.at[pl.ds(b * U8 + REM_BASE, REM)])

        def load_idx(b):
            obase = b * U8
            pltpu.sync_copy(rc_hbm.at[pl.ds(base, CH)], idx_v)

            def sel(i, c):
                off = pl.multiple_of(i * _L, _L)
                ident = base + off + lane
                idx_v[pl.ds(off, _L)] = obase + jnp.where(
                    m, idx_v[pl.ds(off, _L)], ident)
                return c

            lax.fori_loop(0, CH // _L, sel, 0)

        def startg(p):
            size = sizes[p]
            src = lin_hbm.at[idx_v.at[pl.ds(p * P, size)]]
            return pltpu.async_copy(
                src, gbs[p % 2].at[pl.ds(0, size)], semgs[p % 2])

        def g_piece(b, p, copies):
            copies[("g", p)].wait()
            size = sizes[p]
            if p + 2 < np_:
                copies[("g", p + 2)] = startg(p + 2)
            repack(gbs[p % 2], ob, size)
            pltpu.sync_copy(ob.at[pl.ds(0, size)],
                            out_hbm.at[b, pl.ds(base + p * P, size), :])

        def g_rem(b):
            @pl.when(sid == NS - 1)
            def _():
                obase = b * U8
                pltpu.sync_copy(rc_hbm.at[pl.ds(REM_BASE, _L)],
                                idx_v.at[pl.ds(0, _L)])
                ident = REM_BASE + lane
                idx_v[pl.ds(0, _L)] = obase + jnp.where(
                    m, idx_v[pl.ds(0, _L)], ident)
                cp = pltpu.async_copy(
                    lin_hbm.at[idx_v.at[pl.ds(0, REM)]],
                    gbs[0].at[pl.ds(0, REM)], semgs[0])
                cp.wait()
                repack(gbs[0], ob, REM)
                pltpu.sync_copy(ob.at[pl.ds(0, REM)],
                                out_hbm.at[b, pl.ds(REM_BASE, REM), :])

        b0 = cid * BPC
        b1 = b0 + 1

        copies = {("l", 0): startl(b0, 0)}
        if np_ > 1:
            copies[("l", 1)] = startl(b0, 1)
        for p in range(np_):
            l_piece(b0, p, copies)
        if REM:
            l_rem(b0)

        plsc.subcore_barrier()

        load_idx(b0)
        copies = {("l", 0): startl(b1, 0), ("g", 0): startg(0)}
        if np_ > 1:
            copies[("l", 1)] = startl(b1, 1)
            copies[("g", 1)] = startg(1)
        for p in range(np_):
            g_piece(b0, p, copies)
            l_piece(b1, p, copies)
        if REM:
            g_rem(b0)
            l_rem(b1)

        plsc.subcore_barrier()

        load_idx(b1)
        copies = {("g", 0): startg(0)}
        if np_ > 1:
            copies[("g", 1)] = startg(1)
        for p in range(np_):
            g_piece(b1, p, copies)
        if REM:
            g_rem(b1)

    rc = _rc_order_np(U, _DIAGONAL_OFFSET)
    if U_PAD > U:
        rc = np.concatenate([rc, np.zeros(U_PAD - U, np.int32)])
    return sc_gather, jnp.asarray(rc)


def kernel(x_ut, reverse):
    B, U, D = x_ut.shape
    sc_gather, rc = _build(B, U, D)
    rev16 = jnp.broadcast_to(jnp.asarray(reverse, jnp.int32), (_L,))
    return sc_gather(x_ut, rc, rev16)


# speedup vs baseline: 1.0029x; 1.0029x over previous
import functools

import numpy as np
import jax
import jax.numpy as jnp
from jax import lax
from jax.experimental import pallas as pl
from jax.experimental.pallas import tpu as pltpu
from jax.experimental.pallas import tpu_sc as plsc

_DIAGONAL_OFFSET = 2
_L = 16


def _rc_order_np(ut_len: int, diagonal_offset: int) -> np.ndarray:
    seq_len = int(np.sqrt(2 * ut_len + 0.25) - 0.5) + diagonal_offset
    ut_indexes = np.triu_indices(seq_len, diagonal_offset)
    mat_ut_indexes = np.zeros((seq_len, seq_len), dtype=np.int64)
    mat_ut_indexes[ut_indexes] = np.arange(ut_len)
    mask_ut = np.zeros((seq_len, seq_len), dtype=bool)
    mask_ut[ut_indexes] = True
    mat_indexes = mat_ut_indexes + np.multiply(~mask_ut, mat_ut_indexes.T)
    return mat_indexes[::-1, ::-1][ut_indexes].astype(np.int32)


@functools.lru_cache(maxsize=4)
def _build(B: int, U: int, D: int):
    NC, NS = 2, 16
    BPC = B // NC
    U8 = -(-U // 8) * 8
    CH = (U // NS) // 8 * 8
    REM = U - NS * CH
    P = 112
    sizes = [P] * (CH // P) + ([CH % P] if CH % P else [])
    np_ = len(sizes)
    REM_BASE = NS * CH
    U_PAD = U if REM == 0 else REM_BASE + _L
    G = D // _L

    mesh = plsc.VectorSubcoreMesh(core_axis_name="c", subcore_axis_name="s")

    @functools.partial(
        pl.kernel,
        mesh=mesh,
        out_type=jax.ShapeDtypeStruct((B, U, D), jnp.float32),
        scratch_types=[
            pltpu.HBM((B * U8, 2 * D), jnp.float32),
            pltpu.VMEM((CH,), jnp.int32),
            pltpu.VMEM((P, D), jnp.float32),
            pltpu.VMEM((P, D), jnp.float32),
            pltpu.VMEM((P, 2 * D), jnp.float32),
            pltpu.VMEM((P, 2 * D), jnp.float32),
            pltpu.VMEM((P, 2 * D), jnp.float32),
            pltpu.VMEM((P, 2 * D), jnp.float32),
            pltpu.VMEM((P, D), jnp.float32),
            pltpu.VMEM((P, D), jnp.float32),
            pltpu.VMEM((_L,), jnp.int32),
            pltpu.SemaphoreType.DMA,
            pltpu.SemaphoreType.DMA,
            pltpu.SemaphoreType.DMA,
            pltpu.SemaphoreType.DMA,
            pltpu.SemaphoreType.DMA,
            pltpu.SemaphoreType.DMA,
            pltpu.SemaphoreType.DMA,
            pltpu.SemaphoreType.DMA,
        ],
    )
    def sc_gather(x_hbm, rc_hbm, rev_hbm, out_hbm,
                  lin_hbm, idx_v, la0, la1, lb0, lb1, gb0, gb1, ob0, ob1,
                  rev_v,
                  seml0, seml1, semg0, semg1,
                  semwl0, semwl1, semwg0, semwg1):
        cid = lax.axis_index("c")
        sid = lax.axis_index("s")
        base = sid * CH
        las, gbs = (la0, la1), (gb0, gb1)
        lbs, obs = (lb0, lb1), (ob0, ob1)
        semls, semgs = (seml0, seml1), (semg0, semg1)
        semwls, semwgs = (semwl0, semwl1), (semwg0, semwg1)

        pltpu.sync_copy(rev_hbm, rev_v)
        m = rev_v[...] != 0
        lane = lax.iota(jnp.int32, _L)

        def startl(b, p):
            size = sizes[p]
            return pltpu.async_copy(
                x_hbm.at[b, pl.ds(base + p * P, size), :],
                las[p % 2].at[pl.ds(0, size)], semls[p % 2])

        def repack(src, dst, size):
            def body(t, c):
                for g in range(G):
                    dst[t, pl.ds(g * _L, _L)] = src[t, pl.ds(g * _L, _L)]
                return c
            lax.fori_loop(0, size, body, 0)

        def l_piece(b, p, copies):
            copies[("l", p)].wait()
            size = sizes[p]
            if ("wl", p - 2) in copies:
                copies.pop(("wl", p - 2)).wait()
            repack(las[p % 2], lbs[p % 2], size)
            copies[("wl", p)] = pltpu.async_copy(
                lbs[p % 2].at[pl.ds(0, size)],
                lin_hbm.at[pl.ds(b * U8 + base + p * P, size)],
                semwls[p % 2])
            if p + 2 < np_:
                copies[("l", p + 2)] = startl(b, p + 2)

        def l_drain(copies):
            for k in [k for k in copies if k[0] == "wl"]:
                copies.pop(k).wait()

        def l_rem(b):
            @pl.when(sid == NS - 1)
            def _():
                cp = pltpu.async_copy(
                    x_hbm.at[b, pl.ds(REM_BASE, REM), :],
                    las[0].at[pl.ds(0, REM)], semls[0])
                cp.wait()
                repack(las[0], lbs[0], REM)
                pltpu.sync_copy(lbs[0].at[pl.ds(0, REM)],
                                lin_hbm.at[pl.ds(b * U8 + REM_BASE, REM)])

        def load_idx(b):
            obase = b * U8
            pltpu.sync_copy(rc_hbm.at[pl.ds(base, CH)], idx_v)

            def sel(i, c):
                off = pl.multiple_of(i * _L, _L)
                ident = base + off + lane
                idx_v[pl.ds(off, _L)] = obase + jnp.where(
                    m, idx_v[pl.ds(off, _L)], ident)
                return c

            lax.fori_loop(0, CH // _L, sel, 0)

        def startg(p):
            size = sizes[p]
            src = lin_hbm.at[idx_v.at[pl.ds(p * P, size)]]
            return pltpu.async_copy(
                src, gbs[p % 2].at[pl.ds(0, size)], semgs[p % 2])

        def g_piece(b, p, copies):
            copies[("g", p)].wait()
            size = sizes[p]
            if ("wg", p - 2) in copies:
                copies.pop(("wg", p - 2)).wait()
            repack(gbs[p % 2], obs[p % 2], size)
            copies[("wg", p)] = pltpu.async_copy(
                obs[p % 2].at[pl.ds(0, size)],
                out_hbm.at[b, pl.ds(base + p * P, size), :],
                semwgs[p % 2])
            if p + 2 < np_:
                copies[("g", p + 2)] = startg(p + 2)

        def g_drain(copies):
            for k in [k for k in copies if k[0] == "wg"]:
                copies.pop(k).wait()

        def g_rem(b):
            @pl.when(sid == NS - 1)
            def _():
                obase = b * U8
                pltpu.sync_copy(rc_hbm.at[pl.ds(REM_BASE, _L)],
                                idx_v.at[pl.ds(0, _L)])
                ident = REM_BASE + lane
                idx_v[pl.ds(0, _L)] = obase + jnp.where(
                    m, idx_v[pl.ds(0, _L)], ident)
                cp = pltpu.async_copy(
                    lin_hbm.at[idx_v.at[pl.ds(0, REM)]],
                    gbs[0].at[pl.ds(0, REM)], semgs[0])
                cp.wait()
                repack(gbs[0], obs[0], REM)
                pltpu.sync_copy(obs[0].at[pl.ds(0, REM)],
                                out_hbm.at[b, pl.ds(REM_BASE, REM), :])

        b0 = cid * BPC
        b1 = b0 + 1

        copies = {("l", 0): startl(b0, 0)}
        if np_ > 1:
            copies[("l", 1)] = startl(b0, 1)
        for p in range(np_):
            l_piece(b0, p, copies)
        l_drain(copies)
        if REM:
            l_rem(b0)

        plsc.subcore_barrier()

        load_idx(b0)
        copies = {("l", 0): startl(b1, 0), ("g", 0): startg(0)}
        if np_ > 1:
            copies[("l", 1)] = startl(b1, 1)
            copies[("g", 1)] = startg(1)
        for p in range(np_):
            g_piece(b0, p, copies)
            l_piece(b1, p, copies)
        l_drain(copies)
        g_drain(copies)
        if REM:
            g_rem(b0)
            l_rem(b1)

        plsc.subcore_barrier()

        load_idx(b1)
        copies = {("g", 0): startg(0)}
        if np_ > 1:
            copies[("g", 1)] = startg(1)
        for p in range(np_):
            g_piece(b1, p, copies)
        g_drain(copies)
        if REM:
            g_rem(b1)

    rc = _rc_order_np(U, _DIAGONAL_OFFSET)
    if U_PAD > U:
        rc = np.concatenate([rc, np.zeros(U_PAD - U, np.int32)])
    return sc_gather, jnp.asarray(rc)


def kernel(x_ut, reverse):
    B, U, D = x_ut.shape
    sc_gather, rc = _build(B, U, D)
    rev16 = jnp.broadcast_to(jnp.asarray(reverse, jnp.int32), (_L,))
    return sc_gather(x_ut, rc, rev16)


# speedup vs baseline: 1.0092x; 1.0063x over previous
import functools

import numpy as np
import jax
import jax.numpy as jnp
from jax import lax
from jax.experimental import pallas as pl
from jax.experimental.pallas import tpu as pltpu
from jax.experimental.pallas import tpu_sc as plsc

_DIAGONAL_OFFSET = 2
_L = 16


def _rc_order_np(ut_len: int, diagonal_offset: int) -> np.ndarray:
    seq_len = int(np.sqrt(2 * ut_len + 0.25) - 0.5) + diagonal_offset
    ut_indexes = np.triu_indices(seq_len, diagonal_offset)
    mat_ut_indexes = np.zeros((seq_len, seq_len), dtype=np.int64)
    mat_ut_indexes[ut_indexes] = np.arange(ut_len)
    mask_ut = np.zeros((seq_len, seq_len), dtype=bool)
    mask_ut[ut_indexes] = True
    mat_indexes = mat_ut_indexes + np.multiply(~mask_ut, mat_ut_indexes.T)
    return mat_indexes[::-1, ::-1][ut_indexes].astype(np.int32)


@functools.lru_cache(maxsize=4)
def _build(B: int, U: int, D: int):
    NC, NS = 2, 16
    BPC = B // NC
    U8 = -(-U // 8) * 8
    CH = (U // NS) // 8 * 8
    REM = U - NS * CH
    P = 112
    F = CH // P
    TAIL = CH - F * P
    assert F % 2 == 0
    REM_BASE = NS * CH
    U_PAD = U if REM == 0 else REM_BASE + _L
    G = D // _L

    mesh = plsc.VectorSubcoreMesh(core_axis_name="c", subcore_axis_name="s")

    @functools.partial(
        pl.kernel,
        mesh=mesh,
        out_type=jax.ShapeDtypeStruct((B, U, D), jnp.float32),
        scratch_types=[
            pltpu.HBM((B * U8, 2 * D), jnp.float32),
            pltpu.VMEM((CH,), jnp.int32),
            pltpu.VMEM((P, D), jnp.float32),
            pltpu.VMEM((P, D), jnp.float32),
            pltpu.VMEM((P, 2 * D), jnp.float32),
            pltpu.VMEM((P, 2 * D), jnp.float32),
            pltpu.VMEM((P, 2 * D), jnp.float32),
            pltpu.VMEM((P, 2 * D), jnp.float32),
            pltpu.VMEM((P, D), jnp.float32),
            pltpu.VMEM((P, D), jnp.float32),
            pltpu.VMEM((_L,), jnp.int32),
            pltpu.SemaphoreType.DMA,
            pltpu.SemaphoreType.DMA,
            pltpu.SemaphoreType.DMA,
            pltpu.SemaphoreType.DMA,
            pltpu.SemaphoreType.DMA,
            pltpu.SemaphoreType.DMA,
            pltpu.SemaphoreType.DMA,
            pltpu.SemaphoreType.DMA,
        ],
    )
    def sc_gather(x_hbm, rc_hbm, rev_hbm, out_hbm,
                  lin_hbm, idx_v, la0, la1, lb0, lb1, gb0, gb1, ob0, ob1,
                  rev_v,
                  seml0, seml1, semg0, semg1,
                  semwl0, semwl1, semwg0, semwg1):
        cid = lax.axis_index("c")
        sid = lax.axis_index("s")
        base = sid * CH
        las, gbs = (la0, la1), (gb0, gb1)
        lbs, obs = (lb0, lb1), (ob0, ob1)
        semls, semgs = (seml0, seml1), (semg0, semg1)
        semwls, semwgs = (semwl0, semwl1), (semwg0, semwg1)

        pltpu.sync_copy(rev_hbm, rev_v)
        m = rev_v[...] != 0
        lane = lax.iota(jnp.int32, _L)

        def repack(src, dst, size):
            unroll = 4

            def body(t4, c):
                t = t4 * unroll
                for dt in range(unroll):
                    for g in range(G):
                        dst[t + dt, pl.ds(g * _L, _L)] = (
                            src[t + dt, pl.ds(g * _L, _L)])
                return c

            lax.fori_loop(0, size // unroll, body, 0)
            for r in range(size - size % unroll, size):
                for g in range(G):
                    dst[r, pl.ds(g * _L, _L)] = src[r, pl.ds(g * _L, _L)]

        def startl(b, p, j):
            off = pl.multiple_of(base + p * P, 8)
            pltpu.async_copy(x_hbm.at[b, pl.ds(off, P), :], las[j], semls[j])

        def wait_read_l(j):
            pltpu.make_async_copy(
                x_hbm.at[0, pl.ds(0, P), :], las[j], semls[j]).wait()

        def start_write_l(b, p, j):
            off = pl.multiple_of(b * U8 + base + p * P, 8)
            pltpu.async_copy(lbs[j], lin_hbm.at[pl.ds(off, P)], semwls[j])

        def wait_write_l(j):
            pltpu.make_async_copy(
                lbs[j], lin_hbm.at[pl.ds(0, P)], semwls[j]).wait()

        def l_phase(b):
            startl(b, 0, 0)
            startl(b, 1, 1)

            def body(k, c):
                for j in range(2):
                    p = 2 * k + j
                    wait_read_l(j)

                    @pl.when(k > 0)
                    def _():
                        wait_write_l(j)

                    repack(las[j], lbs[j], P)
                    start_write_l(b, p, j)

                    @pl.when(p + 2 < F)
                    def _():
                        startl(b, p + 2, j)
                return c

            lax.fori_loop(0, F // 2, body, 0)
            wait_write_l(0)
            wait_write_l(1)
            if TAIL:
                pltpu.sync_copy(
                    x_hbm.at[b, pl.ds(base + F * P, TAIL), :],
                    las[0].at[pl.ds(0, TAIL)])
                repack(las[0], lbs[0], TAIL)
                pltpu.sync_copy(
                    lbs[0].at[pl.ds(0, TAIL)],
                    lin_hbm.at[pl.ds(b * U8 + base + F * P, TAIL)])
            if REM:
                @pl.when(sid == NS - 1)
                def _():
                    pltpu.sync_copy(
                        x_hbm.at[b, pl.ds(REM_BASE, REM), :],
                        las[0].at[pl.ds(0, REM)])
                    repack(las[0], lbs[0], REM)
                    pltpu.sync_copy(
                        lbs[0].at[pl.ds(0, REM)],
                        lin_hbm.at[pl.ds(b * U8 + REM_BASE, REM)])

        def load_idx(b):
            obase = b * U8
            pltpu.sync_copy(rc_hbm.at[pl.ds(base, CH)], idx_v)

            def sel(i, c):
                off = pl.multiple_of(i * _L, _L)
                ident = base + off + lane
                idx_v[pl.ds(off, _L)] = obase + jnp.where(
                    m, idx_v[pl.ds(off, _L)], ident)
                return c

            lax.fori_loop(0, CH // _L, sel, 0)

        def startg(p, j):
            off = pl.multiple_of(p * P, 8)
            src = lin_hbm.at[idx_v.at[pl.ds(off, P)]]
            pltpu.async_copy(src, gbs[j], semgs[j])

        def wait_read_g(j):
            pltpu.make_async_copy(
                lin_hbm.at[pl.ds(0, P)], gbs[j], semgs[j]).wait()

        def start_write_g(b, p, j):
            off = pl.multiple_of(base + p * P, 8)
            pltpu.async_copy(obs[j], out_hbm.at[b, pl.ds(off, P), :],
                             semwgs[j])

        def wait_write_g(j):
            pltpu.make_async_copy(
                obs[j], out_hbm.at[0, pl.ds(0, P), :], semwgs[j]).wait()

        def g_phase(b):
            load_idx(b)
            startg(0, 0)
            startg(1, 1)

            def body(k, c):
                for j in range(2):
                    p = 2 * k + j
                    wait_read_g(j)

                    @pl.when(k > 0)
                    def _():
                        wait_write_g(j)

                    repack(gbs[j], obs[j], P)
                    start_write_g(b, p, j)

                    @pl.when(p + 2 < F)
                    def _():
                        startg(p + 2, j)
                return c

            lax.fori_loop(0, F // 2, body, 0)
            wait_write_g(0)
            wait_write_g(1)
            if TAIL:
                pltpu.async_copy(
                    lin_hbm.at[idx_v.at[pl.ds(F * P, TAIL)]],
                    gbs[0].at[pl.ds(0, TAIL)], semgs[0]).wait()
                repack(gbs[0], obs[0], TAIL)
                pltpu.sync_copy(
                    obs[0].at[pl.ds(0, TAIL)],
                    out_hbm.at[b, pl.ds(base + F * P, TAIL), :])
            if REM:
                @pl.when(sid == NS - 1)
                def _():
                    obase = b * U8
                    pltpu.sync_copy(rc_hbm.at[pl.ds(REM_BASE, _L)],
                                    idx_v.at[pl.ds(0, _L)])
                    ident = REM_BASE + lane
                    idx_v[pl.ds(0, _L)] = obase + jnp.where(
                        m, idx_v[pl.ds(0, _L)], ident)
                    pltpu.async_copy(
                        lin_hbm.at[idx_v.at[pl.ds(0, REM)]],
                        gbs[0].at[pl.ds(0, REM)], semgs[0]).wait()
                    repack(gbs[0], obs[0], REM)
                    pltpu.sync_copy(
                        obs[0].at[pl.ds(0, REM)],
                        out_hbm.at[b, pl.ds(REM_BASE, REM), :])

        b0 = cid * BPC
        b1 = b0 + 1

        l_phase(b0)
        l_phase(b1)
        plsc.subcore_barrier()
        g_phase(b0)
        g_phase(b1)

    rc = _rc_order_np(U, _DIAGONAL_OFFSET)
    if U_PAD > U:
        rc = np.concatenate([rc, np.zeros(U_PAD - U, np.int32)])
    return sc_gather, jnp.asarray(rc)


def kernel(x_ut, reverse):
    B, U, D = x_ut.shape
    sc_gather, rc = _build(B, U, D)
    rev16 = jnp.broadcast_to(jnp.asarray(reverse, jnp.int32), (_L,))
    return sc_gather(x_ut, rc, rev16)
